# baseline (speedup 1.0000x reference)
import jax
import jax.numpy as jnp
import numpy as np
from jax import lax
from jax.experimental import pallas as pl
from jax.experimental.pallas import tpu as pltpu
from jax.experimental.pallas import tpu_sc as plsc

N_NODES_K = 100000
N_EDGES_K = 6400000
N_GRAPHS_K = 128

NUM_CORES = 2
NUM_SUBCORES = 16
NUM_TILES = NUM_CORES * NUM_SUBCORES
EDGES_PER_TILE = N_EDGES_K // NUM_TILES
CHUNK = 2000
NCHUNKS = EDGES_PER_TILE // CHUNK
INNER = CHUNK // 16

_PERM = np.array([0, 5, 4, 5, 1, 3, 4, 3, 2], dtype=np.int32)
_STRESS_G = np.repeat(np.arange(N_GRAPHS_K, dtype=np.int32), 9)
_STRESS_C = np.tile(_PERM, N_GRAPHS_K).astype(np.int32)


def _rsqrt_len(l2):
    bits = plsc.bitcast(l2, jnp.int32)
    y = plsc.bitcast(jnp.full((16,), 0x5F3759DF, jnp.int32)
                     - lax.shift_right_logical(bits, 1), jnp.float32)
    xhalf = l2 * 0.5
    y = y * (1.5 - xhalf * y * y)
    y = y * (1.5 - xhalf * y * y)
    y = y * (1.5 - xhalf * y * y)
    ln = l2 * y
    return jnp.where(l2 > 0.0, ln, 0.0)


def _body(pos_hbm, ei_hbm, shifts_hbm, ptr_hbm, voigt_hbm, gidx_hbm,
          cidx_hbm, vec_out, len_out, nat_out, stress_out,
          sidx_v, ridx_v, srows_v, rrows_v, shv_v, vecv_v, lenv_v,
          ptr_v, nat_v, voigt_v, gidx_v, cidx_v, stress_v, sem):
    wid = lax.axis_index("s") * NUM_CORES + lax.axis_index("c")
    iota = lax.iota(jnp.int32, 16)
    c0 = jnp.full((16,), 0, jnp.int32)
    c1 = jnp.full((16,), 1, jnp.int32)
    c2 = jnp.full((16,), 2, jnp.int32)

    def chunk_body(j, _):
        base = wid * EDGES_PER_TILE + j * CHUNK
        pltpu.sync_copy(ei_hbm.at[0, pl.ds(base, CHUNK)], sidx_v)
        pltpu.sync_copy(ei_hbm.at[1, pl.ds(base, CHUNK)], ridx_v)
        g1 = pltpu.async_copy(pos_hbm.at[sidx_v], srows_v, sem)
        g2 = pltpu.async_copy(pos_hbm.at[ridx_v], rrows_v, sem)
        pltpu.sync_copy(shifts_hbm.at[pl.ds(base, CHUNK)], shv_v)
        g1.wait()
        g2.wait()

        def edge_body(t, _):
            ii = t * 16 + iota
            vx = (plsc.load_gather(rrows_v, [ii, c0])
                  - plsc.load_gather(srows_v, [ii, c0])
                  + plsc.load_gather(shv_v, [ii, c0]))
            vy = (plsc.load_gather(rrows_v, [ii, c1])
                  - plsc.load_gather(srows_v, [ii, c1])
                  + plsc.load_gather(shv_v, [ii, c1]))
            vz = (plsc.load_gather(rrows_v, [ii, c2])
                  - plsc.load_gather(srows_v, [ii, c2])
                  + plsc.load_gather(shv_v, [ii, c2]))
            plsc.store_scatter(vecv_v, [ii, c0], vx)
            plsc.store_scatter(vecv_v, [ii, c1], vy)
            plsc.store_scatter(vecv_v, [ii, c2], vz)
            l2 = vx * vx + vy * vy + vz * vz
            plsc.store_scatter(lenv_v, [ii, c0], _rsqrt_len(l2))
            return ()

        lax.fori_loop(0, INNER, edge_body, (), unroll=4)
        pltpu.sync_copy(vecv_v, vec_out.at[pl.ds(base, CHUNK)])
        pltpu.sync_copy(lenv_v, len_out.at[pl.ds(base, CHUNK)])
        return ()

    lax.fori_loop(0, NCHUNKS, chunk_body, ())

    @pl.when(wid == 0)
    def _tiny():
        pltpu.sync_copy(ptr_hbm, ptr_v)

        def nat_body(i, _):
            a = plsc.load_gather(ptr_v, [i * 16 + iota])
            b = plsc.load_gather(ptr_v, [i * 16 + 1 + iota])
            nat_v[pl.ds(i * 16, 16)] = b - a
            return ()

        lax.fori_loop(0, N_GRAPHS_K // 16, nat_body, ())
        pltpu.sync_copy(nat_v, nat_out)

        pltpu.sync_copy(voigt_hbm, voigt_v)
        pltpu.sync_copy(gidx_hbm, gidx_v)
        pltpu.sync_copy(cidx_hbm, cidx_v)

        def stress_body(k, _):
            gg = gidx_v[pl.ds(k * 16, 16)]
            cc = cidx_v[pl.ds(k * 16, 16)]
            stress_v[pl.ds(k * 16, 16)] = plsc.load_gather(voigt_v, [gg, cc])
            return ()

        lax.fori_loop(0, (N_GRAPHS_K * 9) // 16, stress_body, ())
        pltpu.sync_copy(stress_v, stress_out)


@jax.jit
def _run(pos8, edge_index, shifts, ptr, voigts, gidx, cidx):
    mesh = plsc.VectorSubcoreMesh(core_axis_name="c", subcore_axis_name="s",
                                  num_cores=NUM_CORES,
                                  num_subcores=NUM_SUBCORES)
    f = pl.kernel(
        _body,
        out_type=[
            jax.ShapeDtypeStruct((N_EDGES_K, 3), jnp.float32),
            jax.ShapeDtypeStruct((N_EDGES_K, 1), jnp.float32),
            jax.ShapeDtypeStruct((N_GRAPHS_K,), jnp.int32),
            jax.ShapeDtypeStruct((N_GRAPHS_K * 9,), jnp.float32),
        ],
        mesh=mesh,
        scratch_types=[
            pltpu.VMEM((CHUNK,), jnp.int32),
            pltpu.VMEM((CHUNK,), jnp.int32),
            pltpu.VMEM((CHUNK, 8), jnp.float32),
            pltpu.VMEM((CHUNK, 8), jnp.float32),
            pltpu.VMEM((CHUNK, 3), jnp.float32),
            pltpu.VMEM((CHUNK, 3), jnp.float32),
            pltpu.VMEM((CHUNK, 1), jnp.float32),
            pltpu.VMEM((N_GRAPHS_K + 1,), jnp.int32),
            pltpu.VMEM((N_GRAPHS_K,), jnp.int32),
            pltpu.VMEM((N_GRAPHS_K, 6), jnp.float32),
            pltpu.VMEM((N_GRAPHS_K * 9,), jnp.int32),
            pltpu.VMEM((N_GRAPHS_K * 9,), jnp.int32),
            pltpu.VMEM((N_GRAPHS_K * 9,), jnp.float32),
            pltpu.SemaphoreType.DMA,
        ],
        compiler_params=pltpu.CompilerParams(needs_layout_passes=False,
                                             use_tc_tiling_on_sc=False),
    )
    return f(pos8, edge_index, shifts, ptr, voigts, gidx, cidx)


def kernel(positions, edge_index, shifts, ptr, voigts):
    pos8 = jnp.pad(positions, ((0, 0), (0, 5)))
    vec, lengths, num_atoms, stress = _run(
        pos8, edge_index.astype(jnp.int32), shifts, ptr.astype(jnp.int32),
        voigts, jnp.asarray(_STRESS_G), jnp.asarray(_STRESS_C))
    return (vec, lengths, num_atoms, stress.reshape(N_GRAPHS_K, 3, 3))


# speedup vs baseline: 13.1364x; 13.1364x over previous
import jax
import jax.numpy as jnp
import numpy as np
from jax import lax
from jax.experimental import pallas as pl
from jax.experimental.pallas import tpu as pltpu
from jax.experimental.pallas import tpu_sc as plsc

N_NODES_K = 100000
N_EDGES_K = 6400000
N_GRAPHS_K = 128

NUM_CORES = 2
NUM_SUBCORES = 16
NUM_TILES = NUM_CORES * NUM_SUBCORES

NBLK = N_EDGES_K // 128
BPC = 16
CHUNK = BPC * 128
NCHUNKS_TOTAL = NBLK // BPC
CHUNKS_PER_TILE = -(-NCHUNKS_TOTAL // NUM_TILES)

_PERM = np.array([0, 5, 4, 5, 1, 3, 4, 3, 2], dtype=np.int32)
_STRESS_G = np.repeat(np.arange(N_GRAPHS_K, dtype=np.int32), 9)
_STRESS_C = np.tile(_PERM, N_GRAPHS_K).astype(np.int32)


def _rsqrt_len(l2):
    bits = plsc.bitcast(l2, jnp.int32)
    y = plsc.bitcast(jnp.full((16,), 0x5F3759DF, jnp.int32)
                     - lax.shift_right_logical(bits, 1), jnp.float32)
    xhalf = l2 * 0.5
    y = y * (1.5 - xhalf * y * y)
    y = y * (1.5 - xhalf * y * y)
    y = y * (1.5 - xhalf * y * y)
    ln = l2 * y
    return jnp.where(l2 > 0.0, ln, 0.0)


def _body(pos_hbm, ei_hbm, shifts_hbm, ptr_hbm, voigt_hbm, gidx_hbm,
          cidx_hbm, vec_out, len_out, nat_out, stress_out,
          idx_v, rows_v, shv_v, vecv_v, lenv_v,
          ptr_v, nat_v, voigt_v, gidx_v, cidx_v, stress_v, sem):
    wid = lax.axis_index("s") * NUM_CORES + lax.axis_index("c")
    iota = lax.iota(jnp.int32, 16)
    c0 = jnp.full((16,), 0, jnp.int32)
    c1 = jnp.full((16,), 1, jnp.int32)
    c2 = jnp.full((16,), 2, jnp.int32)

    def chunk_body(k, _):
        cid = wid + k * NUM_TILES

        @pl.when(cid < NCHUNKS_TOTAL)
        def _():
            blk0 = cid * BPC
            pltpu.sync_copy(ei_hbm.at[pl.ds(blk0 * 256, 2 * CHUNK)], idx_v)
            g = pltpu.async_copy(pos_hbm.at[idx_v], rows_v, sem)
            pltpu.sync_copy(shifts_hbm.at[pl.ds(blk0, BPC)], shv_v)
            g.wait()

            def grp_body(t, _):
                b = t // 8
                l0 = (t % 8) * 16
                jj_s = 256 * b + l0 + iota
                jj_r = jj_s + 128
                vx = (plsc.load_gather(rows_v, [jj_r, c0])
                      - plsc.load_gather(rows_v, [jj_s, c0])
                      + shv_v[b, 0, pl.ds(l0, 16)])
                vy = (plsc.load_gather(rows_v, [jj_r, c1])
                      - plsc.load_gather(rows_v, [jj_s, c1])
                      + shv_v[b, 1, pl.ds(l0, 16)])
                vz = (plsc.load_gather(rows_v, [jj_r, c2])
                      - plsc.load_gather(rows_v, [jj_s, c2])
                      + shv_v[b, 2, pl.ds(l0, 16)])
                vecv_v[b, 0, pl.ds(l0, 16)] = vx
                vecv_v[b, 1, pl.ds(l0, 16)] = vy
                vecv_v[b, 2, pl.ds(l0, 16)] = vz
                vecv_v[b, 3, pl.ds(l0, 16)] = jnp.zeros((16,), jnp.float32)
                l2 = vx * vx + vy * vy + vz * vz
                lenv_v[b, pl.ds(l0, 16)] = _rsqrt_len(l2)
                return ()

            lax.fori_loop(0, 8 * BPC, grp_body, (), unroll=4)
            pltpu.sync_copy(vecv_v, vec_out.at[pl.ds(blk0, BPC)])
            pltpu.sync_copy(lenv_v, len_out.at[pl.ds(blk0, BPC)])

        return ()

    lax.fori_loop(0, CHUNKS_PER_TILE, chunk_body, ())

    @pl.when(wid == 0)
    def _tiny():
        pltpu.sync_copy(ptr_hbm, ptr_v)

        def nat_body(i, _):
            a = plsc.load_gather(ptr_v, [i * 16 + iota])
            b = plsc.load_gather(ptr_v, [i * 16 + 1 + iota])
            nat_v[pl.ds(i * 16, 16)] = b - a
            return ()

        lax.fori_loop(0, N_GRAPHS_K // 16, nat_body, ())
        pltpu.sync_copy(nat_v, nat_out)

        pltpu.sync_copy(voigt_hbm, voigt_v)
        pltpu.sync_copy(gidx_hbm, gidx_v)
        pltpu.sync_copy(cidx_hbm, cidx_v)

        def stress_body(k, _):
            gg = gidx_v[pl.ds(k * 16, 16)]
            cc = cidx_v[pl.ds(k * 16, 16)]
            stress_v[pl.ds(k * 16, 16)] = plsc.load_gather(voigt_v, [gg, cc])
            return ()

        lax.fori_loop(0, (N_GRAPHS_K * 9) // 16, stress_body, ())
        pltpu.sync_copy(stress_v, stress_out)


@jax.jit
def _run(pos8, ei_flat, shifts_blk, ptr, voigts, gidx, cidx):
    mesh = plsc.VectorSubcoreMesh(core_axis_name="c", subcore_axis_name="s",
                                  num_cores=NUM_CORES,
                                  num_subcores=NUM_SUBCORES)
    f = pl.kernel(
        _body,
        out_type=[
            jax.ShapeDtypeStruct((NBLK, 4, 128), jnp.float32),
            jax.ShapeDtypeStruct((NBLK, 128), jnp.float32),
            jax.ShapeDtypeStruct((N_GRAPHS_K,), jnp.int32),
            jax.ShapeDtypeStruct((N_GRAPHS_K * 9,), jnp.float32),
        ],
        mesh=mesh,
        scratch_types=[
            pltpu.VMEM((2 * CHUNK,), jnp.int32),
            pltpu.VMEM((2 * CHUNK, 8), jnp.float32),
            pltpu.VMEM((BPC, 4, 128), jnp.float32),
            pltpu.VMEM((BPC, 4, 128), jnp.float32),
            pltpu.VMEM((BPC, 128), jnp.float32),
            pltpu.VMEM((N_GRAPHS_K + 1,), jnp.int32),
            pltpu.VMEM((N_GRAPHS_K,), jnp.int32),
            pltpu.VMEM((N_GRAPHS_K, 6), jnp.float32),
            pltpu.VMEM((N_GRAPHS_K * 9,), jnp.int32),
            pltpu.VMEM((N_GRAPHS_K * 9,), jnp.int32),
            pltpu.VMEM((N_GRAPHS_K * 9,), jnp.float32),
            pltpu.SemaphoreType.DMA,
        ],
        compiler_params=pltpu.CompilerParams(needs_layout_passes=False,
                                             use_tc_tiling_on_sc=False),
    )
    return f(pos8, ei_flat, shifts_blk, ptr, voigts, gidx, cidx)


def kernel(positions, edge_index, shifts, ptr, voigts):
    pos8 = jnp.pad(positions, ((0, 0), (0, 5)))
    ei_flat = (edge_index.astype(jnp.int32)
               .reshape(2, NBLK, 128).transpose(1, 0, 2).reshape(-1))
    shifts_blk = (jnp.pad(shifts, ((0, 0), (0, 1)))
                  .T.reshape(4, NBLK, 128).transpose(1, 0, 2))
    vec_blk, len_blk, num_atoms, stress = _run(
        pos8, ei_flat, shifts_blk, ptr.astype(jnp.int32), voigts,
        jnp.asarray(_STRESS_G), jnp.asarray(_STRESS_C))
    vec = vec_blk.transpose(0, 2, 1).reshape(N_EDGES_K, 4)[:, :3]
    lengths = len_blk.reshape(N_EDGES_K, 1)
    return (vec, lengths, num_atoms, stress.reshape(N_GRAPHS_K, 3, 3))


# speedup vs baseline: 13.1442x; 1.0006x over previous
import jax
import jax.numpy as jnp
import numpy as np
from jax import lax
from jax.experimental import pallas as pl
from jax.experimental.pallas import tpu as pltpu
from jax.experimental.pallas import tpu_sc as plsc

N_NODES_K = 100000
N_EDGES_K = 6400000
N_GRAPHS_K = 128

NUM_CORES = 2
NUM_SUBCORES = 16
NUM_TILES = NUM_CORES * NUM_SUBCORES

NBLK = N_EDGES_K // 128
BPC = 25
CHUNK = BPC * 128
NCHUNKS_TOTAL = NBLK // BPC
CHUNKS_PER_TILE = -(-NCHUNKS_TOTAL // NUM_TILES)

_PERM = np.array([0, 5, 4, 5, 1, 3, 4, 3, 2], dtype=np.int32)
_STRESS_G = np.repeat(np.arange(N_GRAPHS_K, dtype=np.int32), 9)
_STRESS_C = np.tile(_PERM, N_GRAPHS_K).astype(np.int32)


def _rsqrt_len(l2):
    bits = plsc.bitcast(l2, jnp.int32)
    y = plsc.bitcast(jnp.full((16,), 0x5F3759DF, jnp.int32)
                     - lax.shift_right_logical(bits, 1), jnp.float32)
    xhalf = l2 * 0.5
    y = y * (1.5 - xhalf * y * y)
    y = y * (1.5 - xhalf * y * y)
    y = y * (1.5 - xhalf * y * y)
    ln = l2 * y
    return jnp.where(l2 > 0.0, ln, 0.0)


def _body(pos_hbm, ei_hbm, shifts_hbm, ptr_hbm, voigt_hbm, gidx_hbm,
          cidx_hbm, vec_out, len_out, nat_out, stress_out,
          idx_v, rows_v, shv_v, vecv_v, lenv_v,
          ptr_v, nat_v, voigt_v, gidx_v, cidx_v, stress_v, sem):
    wid = lax.axis_index("s") * NUM_CORES + lax.axis_index("c")
    iota = lax.iota(jnp.int32, 16)
    c0 = jnp.full((16,), 0, jnp.int32)
    c1 = jnp.full((16,), 1, jnp.int32)
    c2 = jnp.full((16,), 2, jnp.int32)

    def chunk_body(k, _):
        cid = wid + k * NUM_TILES

        @pl.when(cid < NCHUNKS_TOTAL)
        def _():
            blk0 = cid * BPC
            pltpu.sync_copy(ei_hbm.at[pl.ds(blk0 * 256, 2 * CHUNK)], idx_v)
            g = pltpu.async_copy(pos_hbm.at[idx_v], rows_v, sem)
            pltpu.sync_copy(shifts_hbm.at[pl.ds(blk0, BPC)], shv_v)
            g.wait()

            def grp_body(t, _):
                b = t // 8
                l0 = (t % 8) * 16
                jj_s = 256 * b + l0 + iota
                jj_r = jj_s + 128
                vx = (plsc.load_gather(rows_v, [jj_r, c0])
                      - plsc.load_gather(rows_v, [jj_s, c0])
                      + shv_v[b, 0, pl.ds(l0, 16)])
                vy = (plsc.load_gather(rows_v, [jj_r, c1])
                      - plsc.load_gather(rows_v, [jj_s, c1])
                      + shv_v[b, 1, pl.ds(l0, 16)])
                vz = (plsc.load_gather(rows_v, [jj_r, c2])
                      - plsc.load_gather(rows_v, [jj_s, c2])
                      + shv_v[b, 2, pl.ds(l0, 16)])
                vecv_v[b, 0, pl.ds(l0, 16)] = vx
                vecv_v[b, 1, pl.ds(l0, 16)] = vy
                vecv_v[b, 2, pl.ds(l0, 16)] = vz
                l2 = vx * vx + vy * vy + vz * vz
                lenv_v[b, pl.ds(l0, 16)] = _rsqrt_len(l2)
                return ()

            lax.fori_loop(0, 8 * BPC, grp_body, (), unroll=8)
            pltpu.sync_copy(vecv_v, vec_out.at[pl.ds(blk0, BPC)])
            pltpu.sync_copy(lenv_v, len_out.at[pl.ds(blk0, BPC)])

        return ()

    lax.fori_loop(0, CHUNKS_PER_TILE, chunk_body, ())

    @pl.when(wid == 0)
    def _tiny():
        pltpu.sync_copy(ptr_hbm, ptr_v)

        def nat_body(i, _):
            a = plsc.load_gather(ptr_v, [i * 16 + iota])
            b = plsc.load_gather(ptr_v, [i * 16 + 1 + iota])
            nat_v[pl.ds(i * 16, 16)] = b - a
            return ()

        lax.fori_loop(0, N_GRAPHS_K // 16, nat_body, ())
        pltpu.sync_copy(nat_v, nat_out)

        pltpu.sync_copy(voigt_hbm, voigt_v)
        pltpu.sync_copy(gidx_hbm, gidx_v)
        pltpu.sync_copy(cidx_hbm, cidx_v)

        def stress_body(k, _):
            gg = gidx_v[pl.ds(k * 16, 16)]
            cc = cidx_v[pl.ds(k * 16, 16)]
            stress_v[pl.ds(k * 16, 16)] = plsc.load_gather(voigt_v, [gg, cc])
            return ()

        lax.fori_loop(0, (N_GRAPHS_K * 9) // 16, stress_body, ())
        pltpu.sync_copy(stress_v, stress_out)


@jax.jit
def _run(pos8, ei_flat, shifts_blk, ptr, voigts, gidx, cidx):
    mesh = plsc.VectorSubcoreMesh(core_axis_name="c", subcore_axis_name="s",
                                  num_cores=NUM_CORES,
                                  num_subcores=NUM_SUBCORES)
    f = pl.kernel(
        _body,
        out_type=[
            jax.ShapeDtypeStruct((NBLK, 4, 128), jnp.float32),
            jax.ShapeDtypeStruct((NBLK, 128), jnp.float32),
            jax.ShapeDtypeStruct((N_GRAPHS_K,), jnp.int32),
            jax.ShapeDtypeStruct((N_GRAPHS_K * 9,), jnp.float32),
        ],
        mesh=mesh,
        scratch_types=[
            pltpu.VMEM((2 * CHUNK,), jnp.int32),
            pltpu.VMEM((2 * CHUNK, 8), jnp.float32),
            pltpu.VMEM((BPC, 4, 128), jnp.float32),
            pltpu.VMEM((BPC, 4, 128), jnp.float32),
            pltpu.VMEM((BPC, 128), jnp.float32),
            pltpu.VMEM((N_GRAPHS_K + 1,), jnp.int32),
            pltpu.VMEM((N_GRAPHS_K,), jnp.int32),
            pltpu.VMEM((N_GRAPHS_K, 6), jnp.float32),
            pltpu.VMEM((N_GRAPHS_K * 9,), jnp.int32),
            pltpu.VMEM((N_GRAPHS_K * 9,), jnp.int32),
            pltpu.VMEM((N_GRAPHS_K * 9,), jnp.float32),
            pltpu.SemaphoreType.DMA,
        ],
        compiler_params=pltpu.CompilerParams(needs_layout_passes=False,
                                             use_tc_tiling_on_sc=False),
    )
    return f(pos8, ei_flat, shifts_blk, ptr, voigts, gidx, cidx)


def kernel(positions, edge_index, shifts, ptr, voigts):
    pos8 = jnp.pad(positions, ((0, 0), (0, 5)))
    ei_flat = (edge_index.astype(jnp.int32)
               .reshape(2, NBLK, 128).transpose(1, 0, 2).reshape(-1))
    shifts_blk = (jnp.pad(shifts, ((0, 0), (0, 1)))
                  .T.reshape(4, NBLK, 128).transpose(1, 0, 2))
    vec_blk, len_blk, num_atoms, stress = _run(
        pos8, ei_flat, shifts_blk, ptr.astype(jnp.int32), voigts,
        jnp.asarray(_STRESS_G), jnp.asarray(_STRESS_C))
    vec = vec_blk.transpose(0, 2, 1).reshape(N_EDGES_K, 4)[:, :3]
    lengths = len_blk.reshape(N_EDGES_K, 1)
    return (vec, lengths, num_atoms, stress.reshape(N_GRAPHS_K, 3, 3))


# speedup vs baseline: 19.6682x; 1.4963x over previous
import jax
import jax.numpy as jnp
import numpy as np
from jax import lax
from jax.experimental import pallas as pl
from jax.experimental.pallas import tpu as pltpu
from jax.experimental.pallas import tpu_sc as plsc

N_NODES_K = 100000
N_EDGES_K = 6400000
N_GRAPHS_K = 128

NUM_CORES = 2
NUM_SUBCORES = 16
NUM_TILES = NUM_CORES * NUM_SUBCORES

NBLK = N_EDGES_K // 128
BPC = 16
CHUNK = BPC * 128
NCHUNKS_TOTAL = NBLK // BPC
CHUNKS_PER_TILE = -(-NCHUNKS_TOTAL // NUM_TILES)
NPAIRS = CHUNKS_PER_TILE // 2

_PERM = np.array([0, 5, 4, 5, 1, 3, 4, 3, 2], dtype=np.int32)
_STRESS_G = np.repeat(np.arange(N_GRAPHS_K, dtype=np.int32), 9)
_STRESS_C = np.tile(_PERM, N_GRAPHS_K).astype(np.int32)


def _rsqrt_len(l2):
    bits = plsc.bitcast(l2, jnp.int32)
    y = plsc.bitcast(jnp.full((16,), 0x5F3759DF, jnp.int32)
                     - lax.shift_right_logical(bits, 1), jnp.float32)
    xhalf = l2 * 0.5
    y = y * (1.5 - xhalf * y * y)
    y = y * (1.5 - xhalf * y * y)
    y = y * (1.5 - xhalf * y * y)
    ln = l2 * y
    return jnp.where(l2 > 0.0, ln, 0.0)


def _body(pos_hbm, ei_hbm, shifts_hbm, ptr_hbm, voigt_hbm, gidx_hbm,
          cidx_hbm, vec_out, len_out, nat_out, stress_out,
          idx0_v, idx1_v, rows0_v, rows1_v, shv0_v, shv1_v,
          vecv0_v, vecv1_v, lenv0_v, lenv1_v,
          ptr_v, nat_v, voigt_v, gidx_v, cidx_v, stress_v,
          sem_g0, sem_g1, sem_w0):
    wid = lax.axis_index("s") * NUM_CORES + lax.axis_index("c")
    iota = lax.iota(jnp.int32, 16)
    c0 = jnp.full((16,), 0, jnp.int32)
    c1 = jnp.full((16,), 1, jnp.int32)
    c2 = jnp.full((16,), 2, jnp.int32)

    def fetch(cid, idx_v, rows_v, shv_v, sem):
        blk0 = cid * BPC
        pltpu.sync_copy(ei_hbm.at[pl.ds(blk0 * 256, 2 * CHUNK)], idx_v)
        pltpu.async_copy(pos_hbm.at[idx_v], rows_v, sem)
        pltpu.sync_copy(shifts_hbm.at[pl.ds(blk0, BPC)], shv_v)

    def compute(rows_v, shv_v, vecv_v, lenv_v):
        def grp_body(t, _):
            b = t // 8
            l0 = (t % 8) * 16
            jj_s = 256 * b + l0 + iota
            jj_r = jj_s + 128
            vx = (plsc.load_gather(rows_v, [jj_r, c0])
                  - plsc.load_gather(rows_v, [jj_s, c0])
                  + shv_v[b, 0, pl.ds(l0, 16)])
            vy = (plsc.load_gather(rows_v, [jj_r, c1])
                  - plsc.load_gather(rows_v, [jj_s, c1])
                  + shv_v[b, 1, pl.ds(l0, 16)])
            vz = (plsc.load_gather(rows_v, [jj_r, c2])
                  - plsc.load_gather(rows_v, [jj_s, c2])
                  + shv_v[b, 2, pl.ds(l0, 16)])
            vecv_v[b, 0, pl.ds(l0, 16)] = vx
            vecv_v[b, 1, pl.ds(l0, 16)] = vy
            vecv_v[b, 2, pl.ds(l0, 16)] = vz
            l2 = vx * vx + vy * vy + vz * vz
            lenv_v[b, pl.ds(l0, 16)] = _rsqrt_len(l2)
            return ()

        lax.fori_loop(0, 8 * BPC, grp_body, (), unroll=8)

    def wait_gather(idx_v, rows_v, sem):
        pltpu.make_async_copy(pos_hbm.at[idx_v], rows_v, sem).wait()

    def wait_wb0():
        pltpu.make_async_copy(vecv0_v, vec_out.at[pl.ds(0, BPC)],
                              sem_w0).wait()
        pltpu.make_async_copy(lenv0_v, len_out.at[pl.ds(0, BPC)],
                              sem_w0).wait()

    fetch(wid, idx0_v, rows0_v, shv0_v, sem_g0)

    def pair_body(m, _):
        cid0 = wid + (2 * m) * NUM_TILES
        cid1 = wid + (2 * m + 1) * NUM_TILES
        cid2 = wid + (2 * m + 2) * NUM_TILES

        @pl.when(m > 0)
        def _():
            wait_wb0()

        wait_gather(idx0_v, rows0_v, sem_g0)

        @pl.when(cid1 < NCHUNKS_TOTAL)
        def _():
            fetch(cid1, idx1_v, rows1_v, shv1_v, sem_g1)

        compute(rows0_v, shv0_v, vecv0_v, lenv0_v)
        blk0 = cid0 * BPC
        pltpu.async_copy(vecv0_v, vec_out.at[pl.ds(blk0, BPC)], sem_w0)
        pltpu.async_copy(lenv0_v, len_out.at[pl.ds(blk0, BPC)], sem_w0)

        @pl.when(cid2 < NCHUNKS_TOTAL)
        def _():
            fetch(cid2, idx0_v, rows0_v, shv0_v, sem_g0)

        @pl.when(cid1 < NCHUNKS_TOTAL)
        def _():
            wait_gather(idx1_v, rows1_v, sem_g1)
            compute(rows1_v, shv1_v, vecv1_v, lenv1_v)
            blk1 = cid1 * BPC
            pltpu.sync_copy(vecv1_v, vec_out.at[pl.ds(blk1, BPC)])
            pltpu.sync_copy(lenv1_v, len_out.at[pl.ds(blk1, BPC)])

        return ()

    lax.fori_loop(0, NPAIRS, pair_body, ())
    wait_wb0()

    @pl.when(wid == 0)
    def _tiny():
        pltpu.sync_copy(ptr_hbm, ptr_v)

        def nat_body(i, _):
            a = plsc.load_gather(ptr_v, [i * 16 + iota])
            b = plsc.load_gather(ptr_v, [i * 16 + 1 + iota])
            nat_v[pl.ds(i * 16, 16)] = b - a
            return ()

        lax.fori_loop(0, N_GRAPHS_K // 16, nat_body, ())
        pltpu.sync_copy(nat_v, nat_out)

        pltpu.sync_copy(voigt_hbm, voigt_v)
        pltpu.sync_copy(gidx_hbm, gidx_v)
        pltpu.sync_copy(cidx_hbm, cidx_v)

        def stress_body(k, _):
            gg = gidx_v[pl.ds(k * 16, 16)]
            cc = cidx_v[pl.ds(k * 16, 16)]
            stress_v[pl.ds(k * 16, 16)] = plsc.load_gather(voigt_v, [gg, cc])
            return ()

        lax.fori_loop(0, (N_GRAPHS_K * 9) // 16, stress_body, ())
        pltpu.sync_copy(stress_v, stress_out)


@jax.jit
def _run(pos8, ei_flat, shifts_blk, ptr, voigts, gidx, cidx):
    mesh = plsc.VectorSubcoreMesh(core_axis_name="c", subcore_axis_name="s",
                                  num_cores=NUM_CORES,
                                  num_subcores=NUM_SUBCORES)
    f = pl.kernel(
        _body,
        out_type=[
            jax.ShapeDtypeStruct((NBLK, 4, 128), jnp.float32),
            jax.ShapeDtypeStruct((NBLK, 128), jnp.float32),
            jax.ShapeDtypeStruct((N_GRAPHS_K,), jnp.int32),
            jax.ShapeDtypeStruct((N_GRAPHS_K * 9,), jnp.float32),
        ],
        mesh=mesh,
        scratch_types=[
            pltpu.VMEM((2 * CHUNK,), jnp.int32),
            pltpu.VMEM((2 * CHUNK,), jnp.int32),
            pltpu.VMEM((2 * CHUNK, 8), jnp.float32),
            pltpu.VMEM((2 * CHUNK, 8), jnp.float32),
            pltpu.VMEM((BPC, 4, 128), jnp.float32),
            pltpu.VMEM((BPC, 4, 128), jnp.float32),
            pltpu.VMEM((BPC, 4, 128), jnp.float32),
            pltpu.VMEM((BPC, 4, 128), jnp.float32),
            pltpu.VMEM((BPC, 128), jnp.float32),
            pltpu.VMEM((BPC, 128), jnp.float32),
            pltpu.VMEM((N_GRAPHS_K + 1,), jnp.int32),
            pltpu.VMEM((N_GRAPHS_K,), jnp.int32),
            pltpu.VMEM((N_GRAPHS_K, 6), jnp.float32),
            pltpu.VMEM((N_GRAPHS_K * 9,), jnp.int32),
            pltpu.VMEM((N_GRAPHS_K * 9,), jnp.int32),
            pltpu.VMEM((N_GRAPHS_K * 9,), jnp.float32),
            pltpu.SemaphoreType.DMA,
            pltpu.SemaphoreType.DMA,
            pltpu.SemaphoreType.DMA,
        ],
        compiler_params=pltpu.CompilerParams(needs_layout_passes=False,
                                             use_tc_tiling_on_sc=False),
    )
    return f(pos8, ei_flat, shifts_blk, ptr, voigts, gidx, cidx)


def kernel(positions, edge_index, shifts, ptr, voigts):
    pos8 = jnp.pad(positions, ((0, 0), (0, 5)))
    ei_flat = (edge_index.astype(jnp.int32)
               .reshape(2, NBLK, 128).transpose(1, 0, 2).reshape(-1))
    shifts_blk = (jnp.pad(shifts, ((0, 0), (0, 1)))
                  .T.reshape(4, NBLK, 128).transpose(1, 0, 2))
    vec_blk, len_blk, num_atoms, stress = _run(
        pos8, ei_flat, shifts_blk, ptr.astype(jnp.int32), voigts,
        jnp.asarray(_STRESS_G), jnp.asarray(_STRESS_C))
    vec = vec_blk.transpose(0, 2, 1).reshape(N_EDGES_K, 4)[:, :3]
    lengths = len_blk.reshape(N_EDGES_K, 1)
    return (vec, lengths, num_atoms, stress.reshape(N_GRAPHS_K, 3, 3))


# speedup vs baseline: 20.0346x; 1.0186x over previous
import jax
import jax.numpy as jnp
import numpy as np
from jax import lax
from jax.experimental import pallas as pl
from jax.experimental.pallas import tpu as pltpu
from jax.experimental.pallas import tpu_sc as plsc

N_NODES_K = 100000
N_EDGES_K = 6400000
N_GRAPHS_K = 128

NUM_CORES = 2
NUM_SUBCORES = 16
NUM_TILES = NUM_CORES * NUM_SUBCORES

NBLK = N_EDGES_K // 128
BPC = 16
CHUNK = BPC * 128
NCHUNKS_TOTAL = NBLK // BPC
CHUNKS_PER_TILE = -(-NCHUNKS_TOTAL // NUM_TILES)
NPAIRS = CHUNKS_PER_TILE // 2

_PERM = np.array([0, 5, 4, 5, 1, 3, 4, 3, 2], dtype=np.int32)
_STRESS_G = np.repeat(np.arange(N_GRAPHS_K, dtype=np.int32), 9)
_STRESS_C = np.tile(_PERM, N_GRAPHS_K).astype(np.int32)


def _rsqrt_len(l2):
    bits = plsc.bitcast(l2, jnp.int32)
    y = plsc.bitcast(jnp.full((16,), 0x5F3759DF, jnp.int32)
                     - lax.shift_right_logical(bits, 1), jnp.float32)
    xhalf = l2 * 0.5
    y = y * (1.5 - xhalf * y * y)
    y = y * (1.5 - xhalf * y * y)
    y = y * (1.5 - xhalf * y * y)
    ln = l2 * y
    return jnp.where(l2 > 0.0, ln, 0.0)


def _body(pos_hbm, ei_hbm, shifts_hbm, ptr_hbm, voigt_hbm, gidx_hbm,
          cidx_hbm, vec_out, len_out, nat_out, stress_out,
          idx0_v, idx1_v, rows0_v, rows1_v, shv0_v, shv1_v,
          vecv0_v, vecv1_v, lenv0_v, lenv1_v,
          ptr_v, nat_v, voigt_v, gidx_v, cidx_v, stress_v,
          sem_g0, sem_g1, sem_w0, sem_w1):
    wid = lax.axis_index("s") * NUM_CORES + lax.axis_index("c")
    iota = lax.iota(jnp.int32, 16)
    c0 = jnp.full((16,), 0, jnp.int32)
    c1 = jnp.full((16,), 1, jnp.int32)
    c2 = jnp.full((16,), 2, jnp.int32)

    def fetch(cid, idx_v, rows_v, shv_v, sem):
        blk0 = cid * BPC
        pltpu.sync_copy(ei_hbm.at[pl.ds(blk0 * 256, 2 * CHUNK)], idx_v)
        pltpu.async_copy(pos_hbm.at[idx_v], rows_v, sem)
        pltpu.sync_copy(shifts_hbm.at[pl.ds(blk0, BPC)], shv_v)

    def compute(rows_v, shv_v, vecv_v, lenv_v):
        def grp_body(t, _):
            b = t // 8
            l0 = (t % 8) * 16
            jj_s = 256 * b + l0 + iota
            jj_r = jj_s + 128
            vx = (plsc.load_gather(rows_v, [jj_r, c0])
                  - plsc.load_gather(rows_v, [jj_s, c0])
                  + shv_v[b, 0, pl.ds(l0, 16)])
            vy = (plsc.load_gather(rows_v, [jj_r, c1])
                  - plsc.load_gather(rows_v, [jj_s, c1])
                  + shv_v[b, 1, pl.ds(l0, 16)])
            vz = (plsc.load_gather(rows_v, [jj_r, c2])
                  - plsc.load_gather(rows_v, [jj_s, c2])
                  + shv_v[b, 2, pl.ds(l0, 16)])
            vecv_v[b, 0, pl.ds(l0, 16)] = vx
            vecv_v[b, 1, pl.ds(l0, 16)] = vy
            vecv_v[b, 2, pl.ds(l0, 16)] = vz
            l2 = vx * vx + vy * vy + vz * vz
            lenv_v[b, pl.ds(l0, 16)] = _rsqrt_len(l2)
            return ()

        lax.fori_loop(0, 8 * BPC, grp_body, (), unroll=8)

    def wait_gather(idx_v, rows_v, sem):
        pltpu.make_async_copy(pos_hbm.at[idx_v], rows_v, sem).wait()

    def wait_wb0():
        pltpu.make_async_copy(vecv0_v, vec_out.at[pl.ds(0, BPC)],
                              sem_w0).wait()
        pltpu.make_async_copy(lenv0_v, len_out.at[pl.ds(0, BPC)],
                              sem_w0).wait()

    def wait_wb1():
        pltpu.make_async_copy(vecv1_v, vec_out.at[pl.ds(0, BPC)],
                              sem_w1).wait()
        pltpu.make_async_copy(lenv1_v, len_out.at[pl.ds(0, BPC)],
                              sem_w1).wait()

    fetch(wid, idx0_v, rows0_v, shv0_v, sem_g0)

    def pair_body(m, _):
        cid0 = wid + (2 * m) * NUM_TILES
        cid1 = wid + (2 * m + 1) * NUM_TILES
        cid2 = wid + (2 * m + 2) * NUM_TILES

        @pl.when(m > 0)
        def _():
            wait_wb0()

        wait_gather(idx0_v, rows0_v, sem_g0)

        @pl.when(cid1 < NCHUNKS_TOTAL)
        def _():
            fetch(cid1, idx1_v, rows1_v, shv1_v, sem_g1)

        compute(rows0_v, shv0_v, vecv0_v, lenv0_v)
        blk0 = cid0 * BPC
        pltpu.async_copy(vecv0_v, vec_out.at[pl.ds(blk0, BPC)], sem_w0)
        pltpu.async_copy(lenv0_v, len_out.at[pl.ds(blk0, BPC)], sem_w0)

        @pl.when(cid2 < NCHUNKS_TOTAL)
        def _():
            fetch(cid2, idx0_v, rows0_v, shv0_v, sem_g0)

        @pl.when(cid1 < NCHUNKS_TOTAL)
        def _():
            wait_gather(idx1_v, rows1_v, sem_g1)

            @pl.when(m > 0)
            def _():
                wait_wb1()

            compute(rows1_v, shv1_v, vecv1_v, lenv1_v)
            blk1 = cid1 * BPC
            pltpu.async_copy(vecv1_v, vec_out.at[pl.ds(blk1, BPC)], sem_w1)
            pltpu.async_copy(lenv1_v, len_out.at[pl.ds(blk1, BPC)], sem_w1)

        return ()

    lax.fori_loop(0, NPAIRS, pair_body, ())
    wait_wb0()
    wait_wb1()

    @pl.when(wid == 0)
    def _tiny():
        pltpu.sync_copy(ptr_hbm, ptr_v)

        def nat_body(i, _):
            a = plsc.load_gather(ptr_v, [i * 16 + iota])
            b = plsc.load_gather(ptr_v, [i * 16 + 1 + iota])
            nat_v[pl.ds(i * 16, 16)] = b - a
            return ()

        lax.fori_loop(0, N_GRAPHS_K // 16, nat_body, ())
        pltpu.sync_copy(nat_v, nat_out)

        pltpu.sync_copy(voigt_hbm, voigt_v)
        pltpu.sync_copy(gidx_hbm, gidx_v)
        pltpu.sync_copy(cidx_hbm, cidx_v)

        def stress_body(k, _):
            gg = gidx_v[pl.ds(k * 16, 16)]
            cc = cidx_v[pl.ds(k * 16, 16)]
            stress_v[pl.ds(k * 16, 16)] = plsc.load_gather(voigt_v, [gg, cc])
            return ()

        lax.fori_loop(0, (N_GRAPHS_K * 9) // 16, stress_body, ())
        pltpu.sync_copy(stress_v, stress_out)


@jax.jit
def _run(pos8, ei_flat, shifts_blk, ptr, voigts, gidx, cidx):
    mesh = plsc.VectorSubcoreMesh(core_axis_name="c", subcore_axis_name="s",
                                  num_cores=NUM_CORES,
                                  num_subcores=NUM_SUBCORES)
    f = pl.kernel(
        _body,
        out_type=[
            jax.ShapeDtypeStruct((NBLK, 4, 128), jnp.float32),
            jax.ShapeDtypeStruct((NBLK, 128), jnp.float32),
            jax.ShapeDtypeStruct((N_GRAPHS_K,), jnp.int32),
            jax.ShapeDtypeStruct((N_GRAPHS_K * 9,), jnp.float32),
        ],
        mesh=mesh,
        scratch_types=[
            pltpu.VMEM((2 * CHUNK,), jnp.int32),
            pltpu.VMEM((2 * CHUNK,), jnp.int32),
            pltpu.VMEM((2 * CHUNK, 8), jnp.float32),
            pltpu.VMEM((2 * CHUNK, 8), jnp.float32),
            pltpu.VMEM((BPC, 4, 128), jnp.float32),
            pltpu.VMEM((BPC, 4, 128), jnp.float32),
            pltpu.VMEM((BPC, 4, 128), jnp.float32),
            pltpu.VMEM((BPC, 4, 128), jnp.float32),
            pltpu.VMEM((BPC, 128), jnp.float32),
            pltpu.VMEM((BPC, 128), jnp.float32),
            pltpu.VMEM((N_GRAPHS_K + 1,), jnp.int32),
            pltpu.VMEM((N_GRAPHS_K,), jnp.int32),
            pltpu.VMEM((N_GRAPHS_K, 6), jnp.float32),
            pltpu.VMEM((N_GRAPHS_K * 9,), jnp.int32),
            pltpu.VMEM((N_GRAPHS_K * 9,), jnp.int32),
            pltpu.VMEM((N_GRAPHS_K * 9,), jnp.float32),
            pltpu.SemaphoreType.DMA,
            pltpu.SemaphoreType.DMA,
            pltpu.SemaphoreType.DMA,
            pltpu.SemaphoreType.DMA,
        ],
        compiler_params=pltpu.CompilerParams(needs_layout_passes=False,
                                             use_tc_tiling_on_sc=False),
    )
    return f(pos8, ei_flat, shifts_blk, ptr, voigts, gidx, cidx)


def kernel(positions, edge_index, shifts, ptr, voigts):
    pos8 = jnp.pad(positions, ((0, 0), (0, 5)))
    ei_flat = (edge_index.astype(jnp.int32)
               .reshape(2, NBLK, 128).transpose(1, 0, 2).reshape(-1))
    shifts_blk = (jnp.pad(shifts, ((0, 0), (0, 1)))
                  .T.reshape(4, NBLK, 128).transpose(1, 0, 2))
    vec_blk, len_blk, num_atoms, stress = _run(
        pos8, ei_flat, shifts_blk, ptr.astype(jnp.int32), voigts,
        jnp.asarray(_STRESS_G), jnp.asarray(_STRESS_C))
    vec = vec_blk.transpose(0, 2, 1).reshape(N_EDGES_K, 4)[:, :3]
    lengths = len_blk.reshape(N_EDGES_K, 1)
    return (vec, lengths, num_atoms, stress.reshape(N_GRAPHS_K, 3, 3))
